# baseline (speedup 1.0000x reference)
import functools

import numpy as np
import jax
import jax.numpy as jnp
from jax import lax
from jax.experimental import pallas as pl
from jax.experimental.pallas import tpu as pltpu
from jax.experimental.pallas import tpu_sc as plsc

_EPS = 1e-5
_MAX_LEN = 512


def _positional_encoding(max_len, d):
    pos = np.arange(max_len, dtype=np.float32)[:, None]
    div = np.exp(np.arange(0, d, 2, dtype=np.float32) * (-np.log(10000.0) / d))
    pe = np.zeros((max_len, d), dtype=np.float32)
    pe[:, 0::2] = np.sin(pos * div)
    pe[:, 1::2] = np.cos(pos * div)
    return pe


@functools.lru_cache(maxsize=None)
def _make_sc_gather_paired(V, D, BT):
    info = plsc.get_sparse_core_info()
    NC, NS = info.num_cores, info.num_subcores
    NW = NC * NS
    BT2 = BT // 2
    assert BT2 % NW == 0
    p_per_w = BT2 // NW
    C = 256
    assert p_per_w % C == 0
    nch = p_per_w // C
    mesh = plsc.VectorSubcoreMesh(core_axis_name="c", subcore_axis_name="s")

    @functools.partial(
        pl.kernel,
        mesh=mesh,
        compiler_params=pltpu.CompilerParams(use_tc_tiling_on_sc=False),
        out_type=jax.ShapeDtypeStruct((BT2, 2 * D), jnp.float32),
        scratch_types=[
            pltpu.VMEM((p_per_w,), jnp.int32),
            pltpu.VMEM((p_per_w,), jnp.int32),
            pltpu.VMEM((C, D), jnp.float32),
            pltpu.VMEM((C, D), jnp.float32),
            pltpu.VMEM((C, D), jnp.float32),
            pltpu.VMEM((C, D), jnp.float32),
            pltpu.SemaphoreType.DMA,
            pltpu.SemaphoreType.DMA,
            pltpu.SemaphoreType.DMA,
            pltpu.SemaphoreType.DMA,
        ],
    )
    def gather(table_hbm, idx_e_hbm, idx_o_hbm, out_hbm,
               idx_ve, idx_vo, be0, bo0, be1, bo1, gs0, gs1, os0, os1):
        wid = lax.axis_index("s") * NC + lax.axis_index("c")
        base = wid * p_per_w
        pltpu.sync_copy(idx_e_hbm.at[pl.ds(base, p_per_w)], idx_ve)
        pltpu.sync_copy(idx_o_hbm.at[pl.ds(base, p_per_w)], idx_vo)
        bufs = ((be0, bo0), (be1, bo1))
        gsems = (gs0, gs1)
        osems = (os0, os1)
        gcp = [None, None]
        ocp = [None, None]

        def issue_gathers(slot, c):
            e = pltpu.async_copy(
                table_hbm.at[idx_ve.at[pl.ds(c * C, C)]], bufs[slot][0], gsems[slot]
            )
            o = pltpu.async_copy(
                table_hbm.at[idx_vo.at[pl.ds(c * C, C)]], bufs[slot][1], gsems[slot]
            )
            gcp[slot] = (e, o)

        issue_gathers(0, 0)
        for c in range(nch):
            i = c & 1
            nxt = c + 1
            if nxt < nch:
                j = nxt & 1
                if ocp[j] is not None:
                    ocp[j][0].wait()
                    ocp[j][1].wait()
                issue_gathers(j, nxt)
            gcp[i][0].wait()
            gcp[i][1].wait()
            row0 = base + c * C
            e = pltpu.async_copy(
                bufs[i][0], out_hbm.at[pl.ds(row0, C), pl.ds(0, D)], osems[i]
            )
            o = pltpu.async_copy(
                bufs[i][1], out_hbm.at[pl.ds(row0, C), pl.ds(D, D)], osems[i]
            )
            ocp[i] = (e, o)
        for pair in ocp:
            if pair is not None:
                pair[0].wait()
                pair[1].wait()

    return gather


@functools.lru_cache(maxsize=None)
def _make_tc_dense_paired(BT, R2, L, D, M):
    W_IN = 2 * D
    LP = L // 2
    S = R2 // LP

    def body(x_ref, pe_ref, we_ref, wo_ref, b_ref, g_ref, be_ref, o_ref):
        x = x_ref[...]
        pe = pe_ref[...]
        if S > 1:
            x = (x.reshape(S, LP, W_IN) + pe[None, :, :]).reshape(R2, W_IN)
        else:
            x = x + pe

        def half(w_ref_h):
            y = lax.dot_general(
                x, w_ref_h[...], (((1,), (1,)), ((), ())),
                preferred_element_type=jnp.float32,
            )
            y = y + b_ref[...]
            mean = jnp.mean(y, axis=1, keepdims=True)
            d = y - mean
            var = jnp.mean(d * d, axis=1, keepdims=True)
            return d * lax.rsqrt(var + _EPS) * g_ref[...] + be_ref[...]

        z_e = half(we_ref)
        z_o = half(wo_ref)
        o_ref[...] = jnp.stack([z_e, z_o], axis=1).reshape(2 * R2, M)

    return pl.pallas_call(
        body,
        grid=(BT // (2 * R2),),
        in_specs=[
            pl.BlockSpec((R2, W_IN), lambda i: (i, 0)),
            pl.BlockSpec((LP, W_IN), lambda i: (0, 0)),
            pl.BlockSpec((M, W_IN), lambda i: (0, 0)),
            pl.BlockSpec((M, W_IN), lambda i: (0, 0)),
            pl.BlockSpec((1, M), lambda i: (0, 0)),
            pl.BlockSpec((1, M), lambda i: (0, 0)),
            pl.BlockSpec((1, M), lambda i: (0, 0)),
        ],
        out_specs=pl.BlockSpec((2 * R2, M), lambda i: (i, 0)),
        out_shape=jax.ShapeDtypeStruct((BT, M), jnp.float32),
    )


def kernel(sequence, table, W, b, gamma, beta):
    B, L = sequence.shape
    V, D = table.shape
    M = W.shape[0]
    BT = B * L
    idx = sequence.reshape(BT)
    idx_e = idx[0::2]
    idx_o = idx[1::2]
    tok_pair = _make_sc_gather_paired(V, D, BT)(table, idx_e, idx_o)

    pe = _positional_encoding(_MAX_LEN, D)[:L]
    pe_pair = np.concatenate([pe[0::2], pe[1::2]], axis=1)
    W_e = jnp.pad(W, ((0, 0), (0, D)))
    W_o = jnp.pad(W, ((0, 0), (D, 0)))
    R2 = 2048
    out = _make_tc_dense_paired(BT, R2, L, D, M)(
        tok_pair,
        jnp.asarray(pe_pair),
        W_e,
        W_o,
        b.reshape(1, M),
        gamma.reshape(1, M),
        beta.reshape(1, M),
    )
    return out.reshape(B, L, M)


# speedup vs baseline: 1.1763x; 1.1763x over previous
import functools

import numpy as np
import jax
import jax.numpy as jnp
from jax import lax
from jax.experimental import pallas as pl
from jax.experimental.pallas import tpu as pltpu
from jax.experimental.pallas import tpu_sc as plsc

_EPS = 1e-5
_MAX_LEN = 512


def _positional_encoding(max_len, d):
    pos = np.arange(max_len, dtype=np.float32)[:, None]
    div = np.exp(np.arange(0, d, 2, dtype=np.float32) * (-np.log(10000.0) / d))
    pe = np.zeros((max_len, d), dtype=np.float32)
    pe[:, 0::2] = np.sin(pos * div)
    pe[:, 1::2] = np.cos(pos * div)
    return pe


@functools.lru_cache(maxsize=None)
def _make_sc_gather_paired(V, D, BT):
    info = plsc.get_sparse_core_info()
    NC, NS = info.num_cores, info.num_subcores
    NW = NC * NS
    BT2 = BT // 2
    assert BT2 % NW == 0
    p_per_w = BT2 // NW
    C = 256
    assert p_per_w % C == 0
    nch = p_per_w // C
    mesh = plsc.VectorSubcoreMesh(core_axis_name="c", subcore_axis_name="s")

    @functools.partial(
        pl.kernel,
        mesh=mesh,
        compiler_params=pltpu.CompilerParams(use_tc_tiling_on_sc=False),
        out_type=jax.ShapeDtypeStruct((BT2, 2 * D), jnp.float32),
        scratch_types=[
            pltpu.VMEM((p_per_w,), jnp.int32),
            pltpu.VMEM((p_per_w,), jnp.int32),
            pltpu.VMEM((C, D), jnp.float32),
            pltpu.VMEM((C, D), jnp.float32),
            pltpu.VMEM((C, D), jnp.float32),
            pltpu.VMEM((C, D), jnp.float32),
            pltpu.SemaphoreType.DMA,
            pltpu.SemaphoreType.DMA,
            pltpu.SemaphoreType.DMA,
            pltpu.SemaphoreType.DMA,
        ],
    )
    def gather(table_hbm, idx_hbm, out_hbm,
               idx_va, idx_vb, ba0, bb0, ba1, bb1, gs0, gs1, os0, os1):
        wid = lax.axis_index("s") * NC + lax.axis_index("c")
        base = wid * p_per_w
        pltpu.sync_copy(idx_hbm.at[pl.ds(base, p_per_w)], idx_va)
        pltpu.sync_copy(idx_hbm.at[pl.ds(BT2 + base, p_per_w)], idx_vb)
        bufs = ((ba0, bb0), (ba1, bb1))
        gsems = (gs0, gs1)
        osems = (os0, os1)
        gcp = [None, None]
        ocp = [None, None]

        def issue_gathers(slot, c):
            a = pltpu.async_copy(
                table_hbm.at[idx_va.at[pl.ds(c * C, C)]], bufs[slot][0], gsems[slot]
            )
            bq = pltpu.async_copy(
                table_hbm.at[idx_vb.at[pl.ds(c * C, C)]], bufs[slot][1], gsems[slot]
            )
            gcp[slot] = (a, bq)

        issue_gathers(0, 0)
        for c in range(nch):
            i = c & 1
            nxt = c + 1
            if nxt < nch:
                j = nxt & 1
                if ocp[j] is not None:
                    ocp[j][0].wait()
                    ocp[j][1].wait()
                issue_gathers(j, nxt)
            gcp[i][0].wait()
            gcp[i][1].wait()
            row0 = base + c * C
            a = pltpu.async_copy(
                bufs[i][0], out_hbm.at[pl.ds(row0, C), pl.ds(0, D)], osems[i]
            )
            bq = pltpu.async_copy(
                bufs[i][1], out_hbm.at[pl.ds(row0, C), pl.ds(D, D)], osems[i]
            )
            ocp[i] = (a, bq)
        for pair in ocp:
            if pair is not None:
                pair[0].wait()
                pair[1].wait()

    return gather


@functools.lru_cache(maxsize=None)
def _make_tc_dense_paired(BT, R2, L, D, M):
    W_IN = 2 * D
    S = R2 // L

    def body(x_ref, pe_ref, wa_ref, wb_ref, b_ref, g_ref, be_ref, o_ref):
        x = x_ref[...]
        pe = pe_ref[...]
        if S > 1:
            x = (x.reshape(S, L, W_IN) + pe[None, :, :]).reshape(R2, W_IN)
        else:
            x = x + pe

        def half(w_ref_h):
            y = lax.dot_general(
                x, w_ref_h[...], (((1,), (1,)), ((), ())),
                preferred_element_type=jnp.float32,
            )
            y = y + b_ref[...]
            mean = jnp.mean(y, axis=1, keepdims=True)
            d = y - mean
            var = jnp.mean(d * d, axis=1, keepdims=True)
            return d * lax.rsqrt(var + _EPS) * g_ref[...] + be_ref[...]

        o_ref[0, :, :] = half(wa_ref)
        o_ref[1, :, :] = half(wb_ref)

    return pl.pallas_call(
        body,
        grid=(BT // (2 * R2),),
        in_specs=[
            pl.BlockSpec((R2, W_IN), lambda i: (i, 0)),
            pl.BlockSpec((L, W_IN), lambda i: (0, 0)),
            pl.BlockSpec((M, W_IN), lambda i: (0, 0)),
            pl.BlockSpec((M, W_IN), lambda i: (0, 0)),
            pl.BlockSpec((1, M), lambda i: (0, 0)),
            pl.BlockSpec((1, M), lambda i: (0, 0)),
            pl.BlockSpec((1, M), lambda i: (0, 0)),
        ],
        out_specs=pl.BlockSpec((2, R2, M), lambda i: (0, i, 0)),
        out_shape=jax.ShapeDtypeStruct((2, BT // 2, M), jnp.float32),
    )


def kernel(sequence, table, W, b, gamma, beta):
    B, L = sequence.shape
    V, D = table.shape
    M = W.shape[0]
    BT = B * L
    idx = sequence.reshape(BT)
    tok_pair = _make_sc_gather_paired(V, D, BT)(table, idx)

    pe = _positional_encoding(_MAX_LEN, D)[:L]
    pe_pair = np.concatenate([pe, pe], axis=1)
    W_a = jnp.pad(W, ((0, 0), (0, D)))
    W_b = jnp.pad(W, ((0, 0), (D, 0)))
    R2 = 2048
    out = _make_tc_dense_paired(BT, R2, L, D, M)(
        tok_pair,
        jnp.asarray(pe_pair),
        W_a,
        W_b,
        b.reshape(1, M),
        gamma.reshape(1, M),
        beta.reshape(1, M),
    )
    return out.reshape(B, L, M)


# speedup vs baseline: 1.2334x; 1.0485x over previous
import functools

import numpy as np
import jax
import jax.numpy as jnp
from jax import lax
from jax.experimental import pallas as pl
from jax.experimental.pallas import tpu as pltpu
from jax.experimental.pallas import tpu_sc as plsc

_EPS = 1e-5
_MAX_LEN = 512


def _positional_encoding(max_len, d):
    pos = np.arange(max_len, dtype=np.float32)[:, None]
    div = np.exp(np.arange(0, d, 2, dtype=np.float32) * (-np.log(10000.0) / d))
    pe = np.zeros((max_len, d), dtype=np.float32)
    pe[:, 0::2] = np.sin(pos * div)
    pe[:, 1::2] = np.cos(pos * div)
    return pe


@functools.lru_cache(maxsize=None)
def _make_sc_gather_paired(V, D, BT):
    info = plsc.get_sparse_core_info()
    NC, NS = info.num_cores, info.num_subcores
    NW = NC * NS
    BT2 = BT // 2
    assert BT2 % NW == 0
    p_per_w = BT2 // NW
    C = 256
    assert p_per_w % C == 0
    nch = p_per_w // C
    mesh = plsc.VectorSubcoreMesh(core_axis_name="c", subcore_axis_name="s")

    @functools.partial(
        pl.kernel,
        mesh=mesh,
        compiler_params=pltpu.CompilerParams(use_tc_tiling_on_sc=False),
        out_type=jax.ShapeDtypeStruct((BT2, 2 * D), jnp.float32),
        scratch_types=[
            pltpu.VMEM((p_per_w,), jnp.int32),
            pltpu.VMEM((p_per_w,), jnp.int32),
            pltpu.VMEM((C, D), jnp.float32),
            pltpu.VMEM((C, D), jnp.float32),
            pltpu.VMEM((C, D), jnp.float32),
            pltpu.VMEM((C, D), jnp.float32),
            pltpu.SemaphoreType.DMA,
            pltpu.SemaphoreType.DMA,
            pltpu.SemaphoreType.DMA,
            pltpu.SemaphoreType.DMA,
        ],
    )
    def gather(table_hbm, idx_hbm, out_hbm,
               idx_va, idx_vb, ba0, bb0, ba1, bb1, gs0, gs1, os0, os1):
        wid = lax.axis_index("s") * NC + lax.axis_index("c")
        base = wid * p_per_w
        pltpu.sync_copy(idx_hbm.at[pl.ds(base, p_per_w)], idx_va)
        pltpu.sync_copy(idx_hbm.at[pl.ds(BT2 + base, p_per_w)], idx_vb)
        bufs = ((ba0, bb0), (ba1, bb1))
        gsems = (gs0, gs1)
        osems = (os0, os1)
        gcp = [None, None]
        ocp = [None, None]

        def issue_gathers(slot, c):
            a = pltpu.async_copy(
                table_hbm.at[idx_va.at[pl.ds(c * C, C)]], bufs[slot][0], gsems[slot]
            )
            bq = pltpu.async_copy(
                table_hbm.at[idx_vb.at[pl.ds(c * C, C)]], bufs[slot][1], gsems[slot]
            )
            gcp[slot] = (a, bq)

        issue_gathers(0, 0)
        for c in range(nch):
            i = c & 1
            nxt = c + 1
            if nxt < nch:
                j = nxt & 1
                if ocp[j] is not None:
                    ocp[j][0].wait()
                    ocp[j][1].wait()
                issue_gathers(j, nxt)
            gcp[i][0].wait()
            gcp[i][1].wait()
            row0 = base + c * C
            a = pltpu.async_copy(
                bufs[i][0], out_hbm.at[pl.ds(row0, C), pl.ds(0, D)], osems[i]
            )
            bq = pltpu.async_copy(
                bufs[i][1], out_hbm.at[pl.ds(row0, C), pl.ds(D, D)], osems[i]
            )
            ocp[i] = (a, bq)
        for pair in ocp:
            if pair is not None:
                pair[0].wait()
                pair[1].wait()

    return gather


@functools.lru_cache(maxsize=None)
def _make_tc_dense_paired(BT, R2, L, D, M):
    W_IN = 2 * D
    S = R2 // L

    def body(x_ref, pe_ref, wa_ref, wb_ref, b_ref, g_ref, be_ref, o_ref):
        x = x_ref[...]
        pe = pe_ref[...]
        if S > 1:
            x = (x.reshape(S, L, W_IN) + pe[None, :, :]).reshape(R2, W_IN)
        else:
            x = x + pe

        def half(w_ref_h):
            y = lax.dot_general(
                x, w_ref_h[...], (((1,), (1,)), ((), ())),
                preferred_element_type=jnp.float32,
            )
            y = y + b_ref[...]
            mean = jnp.mean(y, axis=1, keepdims=True)
            d = y - mean
            var = jnp.mean(d * d, axis=1, keepdims=True)
            return d * lax.rsqrt(var + _EPS) * g_ref[...] + be_ref[...]

        o_ref[0, :, :] = half(wa_ref)
        o_ref[1, :, :] = half(wb_ref)

    return pl.pallas_call(
        body,
        grid=(BT // (2 * R2),),
        in_specs=[
            pl.BlockSpec((R2, W_IN), lambda i: (i, 0)),
            pl.BlockSpec((L, W_IN), lambda i: (0, 0)),
            pl.BlockSpec((M, W_IN), lambda i: (0, 0)),
            pl.BlockSpec((M, W_IN), lambda i: (0, 0)),
            pl.BlockSpec((1, M), lambda i: (0, 0)),
            pl.BlockSpec((1, M), lambda i: (0, 0)),
            pl.BlockSpec((1, M), lambda i: (0, 0)),
        ],
        out_specs=pl.BlockSpec((2, R2, M), lambda i: (0, i, 0)),
        out_shape=jax.ShapeDtypeStruct((2, BT // 2, M), jnp.float32),
    )


def kernel(sequence, table, W, b, gamma, beta):
    B, L = sequence.shape
    V, D = table.shape
    M = W.shape[0]
    BT = B * L
    idx = sequence.reshape(BT)
    tok_pair = _make_sc_gather_paired(V, D, BT)(table, idx)

    pe = _positional_encoding(_MAX_LEN, D)[:L]
    pe_pair = np.concatenate([pe, pe], axis=1)
    W_a = jnp.pad(W, ((0, 0), (0, D)))
    W_b = jnp.pad(W, ((0, 0), (D, 0)))
    R2 = 4096
    out = _make_tc_dense_paired(BT, R2, L, D, M)(
        tok_pair,
        jnp.asarray(pe_pair),
        W_a,
        W_b,
        b.reshape(1, M),
        gamma.reshape(1, M),
        beta.reshape(1, M),
    )
    return out.reshape(B, L, M)


# speedup vs baseline: 1.2556x; 1.0180x over previous
import functools

import numpy as np
import jax
import jax.numpy as jnp
from jax import lax
from jax.experimental import pallas as pl
from jax.experimental.pallas import tpu as pltpu
from jax.experimental.pallas import tpu_sc as plsc

_EPS = 1e-5
_MAX_LEN = 512


def _positional_encoding(max_len, d):
    pos = np.arange(max_len, dtype=np.float32)[:, None]
    div = np.exp(np.arange(0, d, 2, dtype=np.float32) * (-np.log(10000.0) / d))
    pe = np.zeros((max_len, d), dtype=np.float32)
    pe[:, 0::2] = np.sin(pos * div)
    pe[:, 1::2] = np.cos(pos * div)
    return pe


@functools.lru_cache(maxsize=None)
def _make_sc_gather_paired(V, D, BT):
    info = plsc.get_sparse_core_info()
    NC, NS = info.num_cores, info.num_subcores
    NW = NC * NS
    BT2 = BT // 2
    assert BT2 % NW == 0
    p_per_w = BT2 // NW
    C = 256
    assert p_per_w % C == 0
    nch = p_per_w // C
    mesh = plsc.VectorSubcoreMesh(core_axis_name="c", subcore_axis_name="s")

    @functools.partial(
        pl.kernel,
        mesh=mesh,
        compiler_params=pltpu.CompilerParams(use_tc_tiling_on_sc=False),
        out_type=jax.ShapeDtypeStruct((BT2, 2 * D), jnp.float32),
        scratch_types=[
            pltpu.VMEM((p_per_w,), jnp.int32),
            pltpu.VMEM((p_per_w,), jnp.int32),
            pltpu.VMEM((C, D), jnp.float32),
            pltpu.VMEM((C, D), jnp.float32),
            pltpu.VMEM((C, D), jnp.float32),
            pltpu.VMEM((C, D), jnp.float32),
            pltpu.SemaphoreType.DMA,
            pltpu.SemaphoreType.DMA,
            pltpu.SemaphoreType.DMA,
            pltpu.SemaphoreType.DMA,
        ],
    )
    def gather(table_hbm, idx_hbm, out_hbm,
               idx_va, idx_vb, ba0, bb0, ba1, bb1, gs0, gs1, os0, os1):
        wid = lax.axis_index("s") * NC + lax.axis_index("c")
        base = wid * p_per_w
        pltpu.sync_copy(idx_hbm.at[pl.ds(base, p_per_w)], idx_va)
        pltpu.sync_copy(idx_hbm.at[pl.ds(BT2 + base, p_per_w)], idx_vb)
        bufs = ((ba0, bb0), (ba1, bb1))
        gsems = (gs0, gs1)
        osems = (os0, os1)
        gcp = [None, None]
        ocp = [None, None]

        def issue_gathers(slot, c):
            a = pltpu.async_copy(
                table_hbm.at[idx_va.at[pl.ds(c * C, C)]], bufs[slot][0], gsems[slot]
            )
            bq = pltpu.async_copy(
                table_hbm.at[idx_vb.at[pl.ds(c * C, C)]], bufs[slot][1], gsems[slot]
            )
            gcp[slot] = (a, bq)

        issue_gathers(0, 0)
        for c in range(nch):
            i = c & 1
            nxt = c + 1
            if nxt < nch:
                j = nxt & 1
                if ocp[j] is not None:
                    ocp[j][0].wait()
                    ocp[j][1].wait()
                issue_gathers(j, nxt)
            gcp[i][0].wait()
            gcp[i][1].wait()
            row0 = base + c * C
            a = pltpu.async_copy(
                bufs[i][0], out_hbm.at[pl.ds(row0, C), pl.ds(0, D)], osems[i]
            )
            bq = pltpu.async_copy(
                bufs[i][1], out_hbm.at[pl.ds(row0, C), pl.ds(D, D)], osems[i]
            )
            ocp[i] = (a, bq)
        for pair in ocp:
            if pair is not None:
                pair[0].wait()
                pair[1].wait()

    return gather


@functools.lru_cache(maxsize=None)
def _make_tc_dense_paired(BT, R2, L, D, M):
    W_IN = 2 * D
    S = R2 // L

    def body(x_ref, pe_ref, wa_ref, wb_ref, b_ref, g_ref, be_ref, o_ref):
        x = x_ref[...]
        pe = pe_ref[...]
        if S > 1:
            x = (x.reshape(S, L, W_IN) + pe[None, :, :]).reshape(R2, W_IN)
        else:
            x = x + pe

        def half(w_ref_h):
            y = lax.dot_general(
                x, w_ref_h[...], (((1,), (1,)), ((), ())),
                preferred_element_type=jnp.float32,
            )
            y = y + b_ref[...]
            mean = jnp.mean(y, axis=1, keepdims=True)
            d = y - mean
            var = jnp.mean(d * d, axis=1, keepdims=True)
            return d * lax.rsqrt(var + _EPS) * g_ref[...] + be_ref[...]

        o_ref[0, :, :] = half(wa_ref)
        o_ref[1, :, :] = half(wb_ref)

    return pl.pallas_call(
        body,
        grid=(BT // (2 * R2),),
        in_specs=[
            pl.BlockSpec((R2, W_IN), lambda i: (i, 0)),
            pl.BlockSpec((L, W_IN), lambda i: (0, 0)),
            pl.BlockSpec((M, W_IN), lambda i: (0, 0)),
            pl.BlockSpec((M, W_IN), lambda i: (0, 0)),
            pl.BlockSpec((1, M), lambda i: (0, 0)),
            pl.BlockSpec((1, M), lambda i: (0, 0)),
            pl.BlockSpec((1, M), lambda i: (0, 0)),
        ],
        out_specs=pl.BlockSpec((2, R2, M), lambda i: (0, i, 0)),
        out_shape=jax.ShapeDtypeStruct((2, BT // 2, M), jnp.float32),
    )


def kernel(sequence, table, W, b, gamma, beta):
    B, L = sequence.shape
    V, D = table.shape
    M = W.shape[0]
    BT = B * L
    idx = sequence.reshape(BT)
    tok_pair = _make_sc_gather_paired(V, D, BT)(table, idx)

    pe = _positional_encoding(_MAX_LEN, D)[:L]
    pe_pair = np.concatenate([pe, pe], axis=1)
    W_a = jnp.pad(W, ((0, 0), (0, D)))
    W_b = jnp.pad(W, ((0, 0), (D, 0)))
    R2 = 8192
    out = _make_tc_dense_paired(BT, R2, L, D, M)(
        tok_pair,
        jnp.asarray(pe_pair),
        W_a,
        W_b,
        b.reshape(1, M),
        gamma.reshape(1, M),
        beta.reshape(1, M),
    )
    return out.reshape(B, L, M)


# speedup vs baseline: 1.3011x; 1.0363x over previous
import functools

import numpy as np
import jax
import jax.numpy as jnp
from jax import lax
from jax.experimental import pallas as pl
from jax.experimental.pallas import tpu as pltpu
from jax.experimental.pallas import tpu_sc as plsc

_EPS = 1e-5
_MAX_LEN = 512


def _positional_encoding(max_len, d):
    pos = np.arange(max_len, dtype=np.float32)[:, None]
    div = np.exp(np.arange(0, d, 2, dtype=np.float32) * (-np.log(10000.0) / d))
    pe = np.zeros((max_len, d), dtype=np.float32)
    pe[:, 0::2] = np.sin(pos * div)
    pe[:, 1::2] = np.cos(pos * div)
    return pe


@functools.lru_cache(maxsize=None)
def _make_sc_gather_paired(V, D, BT):
    info = plsc.get_sparse_core_info()
    NC, NS = info.num_cores, info.num_subcores
    NW = NC * NS
    BT2 = BT // 2
    assert BT2 % NW == 0
    p_per_w = BT2 // NW
    C = 256
    assert p_per_w % C == 0
    nch = p_per_w // C
    mesh = plsc.VectorSubcoreMesh(core_axis_name="c", subcore_axis_name="s")

    @functools.partial(
        pl.kernel,
        mesh=mesh,
        compiler_params=pltpu.CompilerParams(use_tc_tiling_on_sc=False),
        out_type=jax.ShapeDtypeStruct((BT2, 2 * D), jnp.float32),
        scratch_types=[
            pltpu.VMEM((p_per_w,), jnp.int32),
            pltpu.VMEM((p_per_w,), jnp.int32),
            pltpu.VMEM((C, D), jnp.float32),
            pltpu.VMEM((C, D), jnp.float32),
            pltpu.VMEM((C, D), jnp.float32),
            pltpu.VMEM((C, D), jnp.float32),
            pltpu.SemaphoreType.DMA,
            pltpu.SemaphoreType.DMA,
            pltpu.SemaphoreType.DMA,
            pltpu.SemaphoreType.DMA,
        ],
    )
    def gather(table_hbm, idx_hbm, out_hbm,
               idx_va, idx_vb, ba0, bb0, ba1, bb1, gs0, gs1, os0, os1):
        wid = lax.axis_index("s") * NC + lax.axis_index("c")
        base = wid * p_per_w
        pltpu.sync_copy(idx_hbm.at[pl.ds(base, p_per_w)], idx_va)
        pltpu.sync_copy(idx_hbm.at[pl.ds(BT2 + base, p_per_w)], idx_vb)
        bufs = ((ba0, bb0), (ba1, bb1))
        gsems = (gs0, gs1)
        osems = (os0, os1)
        gcp = [None, None]
        ocp = [None, None]

        def issue_gathers(slot, c):
            a = pltpu.async_copy(
                table_hbm.at[idx_va.at[pl.ds(c * C, C)]], bufs[slot][0], gsems[slot]
            )
            bq = pltpu.async_copy(
                table_hbm.at[idx_vb.at[pl.ds(c * C, C)]], bufs[slot][1], gsems[slot]
            )
            gcp[slot] = (a, bq)

        issue_gathers(0, 0)
        for c in range(nch):
            i = c & 1
            nxt = c + 1
            if nxt < nch:
                j = nxt & 1
                if ocp[j] is not None:
                    ocp[j][0].wait()
                    ocp[j][1].wait()
                issue_gathers(j, nxt)
            gcp[i][0].wait()
            gcp[i][1].wait()
            row0 = base + c * C
            a = pltpu.async_copy(
                bufs[i][0], out_hbm.at[pl.ds(row0, C), pl.ds(0, D)], osems[i]
            )
            bq = pltpu.async_copy(
                bufs[i][1], out_hbm.at[pl.ds(row0, C), pl.ds(D, D)], osems[i]
            )
            ocp[i] = (a, bq)
        for pair in ocp:
            if pair is not None:
                pair[0].wait()
                pair[1].wait()

    return gather


@functools.lru_cache(maxsize=None)
def _make_tc_dense_paired(BT, R2, L, D, M):
    W_IN = 2 * D
    S = R2 // L

    def body(x_ref, pe_ref, wa_ref, wb_ref, b_ref, g_ref, be_ref, o_ref):
        x = x_ref[...]
        pe = pe_ref[...]
        if S > 1:
            x = (x.reshape(S, L, W_IN) + pe[None, :, :]).reshape(R2, W_IN)
        else:
            x = x + pe

        ones = jnp.ones((M, M), dtype=jnp.float32)

        def half(w_ref_h):
            d = lax.dot_general(
                x, w_ref_h[...], (((1,), (1,)), ((), ())),
                preferred_element_type=jnp.float32,
            )
            d = d + b_ref[...]
            s2 = lax.dot_general(
                d * d, ones, (((1,), (0,)), ((), ())),
                preferred_element_type=jnp.float32,
            )
            return d * lax.rsqrt(s2 * (1.0 / M) + _EPS) * g_ref[...] + be_ref[...]

        o_ref[0, :, :] = half(wa_ref)
        o_ref[1, :, :] = half(wb_ref)

    return pl.pallas_call(
        body,
        grid=(BT // (2 * R2),),
        in_specs=[
            pl.BlockSpec((R2, W_IN), lambda i: (i, 0)),
            pl.BlockSpec((L, W_IN), lambda i: (0, 0)),
            pl.BlockSpec((M, W_IN), lambda i: (0, 0)),
            pl.BlockSpec((M, W_IN), lambda i: (0, 0)),
            pl.BlockSpec((1, M), lambda i: (0, 0)),
            pl.BlockSpec((1, M), lambda i: (0, 0)),
            pl.BlockSpec((1, M), lambda i: (0, 0)),
        ],
        out_specs=pl.BlockSpec((2, R2, M), lambda i: (0, i, 0)),
        out_shape=jax.ShapeDtypeStruct((2, BT // 2, M), jnp.float32),
    )


def kernel(sequence, table, W, b, gamma, beta):
    B, L = sequence.shape
    V, D = table.shape
    M = W.shape[0]
    BT = B * L
    idx = sequence.reshape(BT)
    tok_pair = _make_sc_gather_paired(V, D, BT)(table, idx)

    pe = _positional_encoding(_MAX_LEN, D)[:L]
    pe_pair = np.concatenate([pe, pe], axis=1)
    Wc = W - jnp.mean(W, axis=0, keepdims=True)
    bc = b - jnp.mean(b)
    W_a = jnp.pad(Wc, ((0, 0), (0, D)))
    W_b = jnp.pad(Wc, ((0, 0), (D, 0)))
    R2 = 8192
    out = _make_tc_dense_paired(BT, R2, L, D, M)(
        tok_pair,
        jnp.asarray(pe_pair),
        W_a,
        W_b,
        bc.reshape(1, M),
        gamma.reshape(1, M),
        beta.reshape(1, M),
    )
    return out.reshape(B, L, M)


# speedup vs baseline: 1.3094x; 1.0064x over previous
import functools

import numpy as np
import jax
import jax.numpy as jnp
from jax import lax
from jax.experimental import pallas as pl
from jax.experimental.pallas import tpu as pltpu
from jax.experimental.pallas import tpu_sc as plsc

_EPS = 1e-5
_MAX_LEN = 512


def _positional_encoding(max_len, d):
    pos = np.arange(max_len, dtype=np.float32)[:, None]
    div = np.exp(np.arange(0, d, 2, dtype=np.float32) * (-np.log(10000.0) / d))
    pe = np.zeros((max_len, d), dtype=np.float32)
    pe[:, 0::2] = np.sin(pos * div)
    pe[:, 1::2] = np.cos(pos * div)
    return pe


@functools.lru_cache(maxsize=None)
def _make_sc_gather_paired(V, D, BT):
    info = plsc.get_sparse_core_info()
    NC, NS = info.num_cores, info.num_subcores
    NW = NC * NS
    BT2 = BT // 2
    assert BT2 % NW == 0
    p_per_w = BT2 // NW
    C = 256
    assert p_per_w % C == 0
    nch = p_per_w // C
    mesh = plsc.VectorSubcoreMesh(core_axis_name="c", subcore_axis_name="s")

    @functools.partial(
        pl.kernel,
        mesh=mesh,
        compiler_params=pltpu.CompilerParams(use_tc_tiling_on_sc=False),
        out_type=jax.ShapeDtypeStruct((BT2, 2 * D), jnp.float32),
        scratch_types=[
            pltpu.VMEM((p_per_w,), jnp.int32),
            pltpu.VMEM((p_per_w,), jnp.int32),
            pltpu.VMEM((C, D), jnp.float32),
            pltpu.VMEM((C, D), jnp.float32),
            pltpu.VMEM((C, D), jnp.float32),
            pltpu.VMEM((C, D), jnp.float32),
            pltpu.SemaphoreType.DMA,
            pltpu.SemaphoreType.DMA,
            pltpu.SemaphoreType.DMA,
            pltpu.SemaphoreType.DMA,
        ],
    )
    def gather(table_hbm, idx_hbm, out_hbm,
               idx_va, idx_vb, ba0, bb0, ba1, bb1, gs0, gs1, os0, os1):
        wid = lax.axis_index("s") * NC + lax.axis_index("c")
        base = wid * p_per_w
        pltpu.sync_copy(idx_hbm.at[pl.ds(base, p_per_w)], idx_va)
        pltpu.sync_copy(idx_hbm.at[pl.ds(BT2 + base, p_per_w)], idx_vb)
        bufs = ((ba0, bb0), (ba1, bb1))
        gsems = (gs0, gs1)
        osems = (os0, os1)
        gcp = [None, None]
        ocp = [None, None]

        def issue_gathers(slot, c):
            a = pltpu.async_copy(
                table_hbm.at[idx_va.at[pl.ds(c * C, C)]], bufs[slot][0], gsems[slot]
            )
            bq = pltpu.async_copy(
                table_hbm.at[idx_vb.at[pl.ds(c * C, C)]], bufs[slot][1], gsems[slot]
            )
            gcp[slot] = (a, bq)

        issue_gathers(0, 0)
        for c in range(nch):
            i = c & 1
            nxt = c + 1
            if nxt < nch:
                j = nxt & 1
                if ocp[j] is not None:
                    ocp[j][0].wait()
                    ocp[j][1].wait()
                issue_gathers(j, nxt)
            gcp[i][0].wait()
            gcp[i][1].wait()
            row0 = base + c * C
            a = pltpu.async_copy(
                bufs[i][0], out_hbm.at[pl.ds(row0, C), pl.ds(0, D)], osems[i]
            )
            bq = pltpu.async_copy(
                bufs[i][1], out_hbm.at[pl.ds(row0, C), pl.ds(D, D)], osems[i]
            )
            ocp[i] = (a, bq)
        for pair in ocp:
            if pair is not None:
                pair[0].wait()
                pair[1].wait()

    return gather


@functools.lru_cache(maxsize=None)
def _make_tc_dense_paired(BT, R2, L, D, M):
    W_IN = 2 * D
    S = R2 // L

    def body(x_ref, pe_ref, wa_ref, wb_ref, b_ref, g_ref, be_ref, o_ref):
        x = x_ref[...]
        pe = pe_ref[...]
        if S > 1:
            x = (x.reshape(S, L, W_IN) + pe[None, :, :]).reshape(R2, W_IN)
        else:
            x = x + pe

        ones = jnp.ones((M, M), dtype=jnp.float32)

        def half(w_ref_h):
            d = lax.dot_general(
                x, w_ref_h[...], (((1,), (1,)), ((), ())),
                preferred_element_type=jnp.float32,
            )
            d = d + b_ref[...]
            s2 = lax.dot_general(
                d * d, ones, (((1,), (0,)), ((), ())),
                preferred_element_type=jnp.float32,
            )
            return d * lax.rsqrt(s2 * (1.0 / M) + _EPS) * g_ref[...] + be_ref[...]

        o_ref[0, :, :] = half(wa_ref)
        o_ref[1, :, :] = half(wb_ref)

    return pl.pallas_call(
        body,
        grid=(BT // (2 * R2),),
        in_specs=[
            pl.BlockSpec((R2, W_IN), lambda i: (i, 0)),
            pl.BlockSpec((L, W_IN), lambda i: (0, 0)),
            pl.BlockSpec((M, W_IN), lambda i: (0, 0)),
            pl.BlockSpec((M, W_IN), lambda i: (0, 0)),
            pl.BlockSpec((1, M), lambda i: (0, 0)),
            pl.BlockSpec((1, M), lambda i: (0, 0)),
            pl.BlockSpec((1, M), lambda i: (0, 0)),
        ],
        out_specs=pl.BlockSpec((2, R2, M), lambda i: (0, i, 0)),
        out_shape=jax.ShapeDtypeStruct((2, BT // 2, M), jnp.float32),
    )


def kernel(sequence, table, W, b, gamma, beta):
    B, L = sequence.shape
    V, D = table.shape
    M = W.shape[0]
    BT = B * L
    idx = sequence.reshape(BT)
    tok_pair = _make_sc_gather_paired(V, D, BT)(table, idx)

    pe = _positional_encoding(_MAX_LEN, D)[:L]
    pe_pair = np.concatenate([pe, pe], axis=1)
    Wc = W - jnp.mean(W, axis=0, keepdims=True)
    bc = b - jnp.mean(b)
    W_a = jnp.pad(Wc, ((0, 0), (0, D)))
    W_b = jnp.pad(Wc, ((0, 0), (D, 0)))
    R2 = 16384
    out = _make_tc_dense_paired(BT, R2, L, D, M)(
        tok_pair,
        jnp.asarray(pe_pair),
        W_a,
        W_b,
        bc.reshape(1, M),
        gamma.reshape(1, M),
        beta.reshape(1, M),
    )
    return out.reshape(B, L, M)
